# baseline (speedup 1.0000x reference)
import jax
import jax.numpy as jnp
from jax import lax
from jax.experimental import pallas as pl
from jax.experimental.pallas import tpu as pltpu

_NBUF = 8


def _make_body(q, k, d, bq):
    num = q // bq

    def body(x_ref, yt_ref, o_hbm, buf, sems):
        s = pl.program_id(0)
        b = lax.rem(s, _NBUF)

        @pl.when(s >= _NBUF)
        def _():
            pltpu.make_async_copy(
                buf.at[b], o_hbm.at[pl.ds((s - _NBUF) * bq, bq)], sems.at[b]
            ).wait()

        buf[b] = x_ref[...][:, :, None] + yt_ref[...][None, :, :]
        pltpu.make_async_copy(
            buf.at[b], o_hbm.at[pl.ds(s * bq, bq)], sems.at[b]
        ).start()

        @pl.when(s == num - 1)
        def _():
            for off in range(min(_NBUF, num)):
                step = num - 1 - off
                pltpu.make_async_copy(
                    buf.at[step % _NBUF],
                    o_hbm.at[pl.ds(step * bq, bq)],
                    sems.at[step % _NBUF],
                ).wait()

    return body, num


def kernel(query_size, key_size, x_emb, y_emb):
    q, d = x_emb.shape
    k, _ = y_emb.shape
    x_eff = jnp.take(x_emb, jnp.arange(q) + (query_size - q), axis=0)
    y_eff = jnp.take(y_emb, jnp.arange(k) + (key_size - k), axis=0)

    yt = y_eff.T
    bq = 16
    body, num = _make_body(q, k, d, bq)
    out3 = pl.pallas_call(
        body,
        grid=(num,),
        in_specs=[
            pl.BlockSpec((bq, d), lambda i: (i, 0)),
            pl.BlockSpec((d, k), lambda i: (0, 0)),
        ],
        out_specs=pl.BlockSpec(memory_space=pl.ANY),
        out_shape=jax.ShapeDtypeStruct((q, d, k), x_emb.dtype),
        scratch_shapes=[
            pltpu.VMEM((_NBUF, bq, d, k), x_emb.dtype),
            pltpu.SemaphoreType.DMA((_NBUF,)),
        ],
    )(x_eff, yt)
    return jnp.transpose(out3, (0, 2, 1))


# speedup vs baseline: 1.0505x; 1.0505x over previous
import jax
import jax.numpy as jnp
from jax.experimental import pallas as pl


def _outer_sum_kernel(x_ref, yt_ref, o_ref):
    o_ref[...] = x_ref[...][:, :, None] + yt_ref[...][None, :, :]


def kernel(query_size, key_size, x_emb, y_emb):
    q, d = x_emb.shape
    k, _ = y_emb.shape
    x_eff = jnp.take(x_emb, jnp.arange(q) + (query_size - q), axis=0)
    y_eff = jnp.take(y_emb, jnp.arange(k) + (key_size - k), axis=0)

    yt = y_eff.T
    bq = 32
    out3 = pl.pallas_call(
        _outer_sum_kernel,
        grid=(q // bq,),
        in_specs=[
            pl.BlockSpec((bq, d), lambda i: (i, 0)),
            pl.BlockSpec((d, k), lambda i: (0, 0)),
        ],
        out_specs=pl.BlockSpec((bq, d, k), lambda i: (i, 0, 0)),
        out_shape=jax.ShapeDtypeStruct((q, d, k), x_emb.dtype),
    )(x_eff, yt)
    return jnp.transpose(out3, (0, 2, 1))
